# baseline (speedup 1.0000x reference)
import functools

import jax
import jax.numpy as jnp
from jax import lax
from jax.experimental import pallas as pl
from jax.experimental.pallas import tpu as pltpu
from jax.experimental.pallas import tpu_sc as plsc

N, E = 10000, 320000
NPAD = 10240
F0, F1, F2, F3, BN = 128, 128, 64, 32, 16

K = 125
CHUNKS = E // K
NC, NS = 2, 16
NW = NC * NS
CPW = CHUNKS // NW
RPT = NPAD // NS
G = 16

_MESH = plsc.VectorSubcoreMesh(core_axis_name="c", subcore_axis_name="s")


DEG_K = 500
DEG_CPW = (E // DEG_K) // NW


def _deg_body(dst_hbm, ones_hbm, zeros_hbm, out_hbm, idx_v, ones_v, acc_sh, sem):
    c = lax.axis_index("c")
    s = lax.axis_index("s")
    wid = c * NS + s
    pltpu.sync_copy(dst_hbm.at[pl.ds(wid * DEG_CPW, DEG_CPW)], idx_v)
    pltpu.sync_copy(ones_hbm, ones_v)
    pltpu.sync_copy(zeros_hbm, acc_sh.at[pl.ds(s * RPT, RPT)])
    plsc.subcore_barrier()

    def body(j, carry):
        pltpu.async_copy(ones_v, acc_sh.at[idx_v.at[j]], sem, add=True)
        return carry

    lax.fori_loop(0, DEG_CPW, body, 0)

    def drain(j, carry):
        pltpu.make_async_copy(ones_v, acc_sh.at[idx_v.at[0]], sem).wait()
        return carry

    lax.fori_loop(0, DEG_CPW, drain, 0)
    plsc.subcore_barrier()
    pltpu.sync_copy(acc_sh.at[pl.ds(s * RPT, RPT)],
                    out_hbm.at[c, pl.ds(s * RPT, RPT)])


def _deg_pass(edge_index, ones, zeros):
    dst2d = edge_index[1].reshape(E // DEG_K, DEG_K)
    return pl.kernel(
        _deg_body,
        out_type=jax.ShapeDtypeStruct((NC, NPAD, 16), jnp.float32),
        mesh=_MESH,
        scratch_types=[
            pltpu.VMEM((DEG_CPW, DEG_K), jnp.int32),
            pltpu.VMEM((DEG_K, 16), jnp.float32),
            pltpu.VMEM_SHARED((NPAD, 16), jnp.float32),
            pltpu.SemaphoreType.DMA,
        ],
        compiler_params=pltpu.CompilerParams(use_tc_tiling_on_sc=False),
    )(dst2d, ones, zeros)


def _scatter_body(k, g, cpw, h_hbm, src_hbm, dst_hbm, zeros_hbm, out_hbm,
                  srcv, dstv, b0, b1, acc_sh, g0, g1, s0, s1):
    c = lax.axis_index("c")
    s = lax.axis_index("s")
    wid = c * NS + s

    @pl.when(c == 0)
    def _():
        pltpu.sync_copy(h_hbm.at[pl.ds(s * RPT, RPT)],
                        acc_sh.at[pl.ds(s * RPT, RPT)])

    @pl.when(c == 1)
    def _():
        pltpu.sync_copy(zeros_hbm, acc_sh.at[pl.ds(s * RPT, RPT)])

    plsc.subcore_barrier()

    def group(gi, carry):
        base = wid * cpw + gi * g
        pltpu.sync_copy(src_hbm.at[pl.ds(base, g)], srcv)
        pltpu.sync_copy(dst_hbm.at[pl.ds(base, g)], dstv)
        pltpu.async_copy(h_hbm.at[srcv.at[0]], b0, g0)
        pltpu.async_copy(h_hbm.at[srcv.at[1]], b1, g1)

        def step(u, carry2):
            k = 2 * u
            pltpu.make_async_copy(h_hbm.at[srcv.at[k]], b0, g0).wait()
            pltpu.async_copy(b0, acc_sh.at[dstv.at[k]], s0, add=True)
            pltpu.make_async_copy(h_hbm.at[srcv.at[k + 1]], b1, g1).wait()
            pltpu.async_copy(b1, acc_sh.at[dstv.at[k + 1]], s1, add=True)

            pltpu.make_async_copy(b0, acc_sh.at[dstv.at[k]], s0).wait()

            @pl.when(k + 2 < g)
            def _():
                pltpu.async_copy(h_hbm.at[srcv.at[k + 2]], b0, g0)

            pltpu.make_async_copy(b1, acc_sh.at[dstv.at[k + 1]], s1).wait()

            @pl.when(k + 3 < g)
            def _():
                pltpu.async_copy(h_hbm.at[srcv.at[k + 3]], b1, g1)

            return carry2

        lax.fori_loop(0, g // 2, step, 0)
        return carry

    lax.fori_loop(0, cpw // g, group, 0)
    plsc.subcore_barrier()
    pltpu.sync_copy(acc_sh.at[pl.ds(s * RPT, RPT)],
                    out_hbm.at[c, pl.ds(s * RPT, RPT)])


def _scatter_pass(h, edge_index, zeros, C, k, g):
    chunks = E // k
    cpw = chunks // NW
    src2d = edge_index[0].reshape(chunks, k)
    dst2d = edge_index[1].reshape(chunks, k)
    return pl.kernel(
        functools.partial(_scatter_body, k, g, cpw),
        out_type=jax.ShapeDtypeStruct((NC, NPAD, C), jnp.float32),
        mesh=_MESH,
        scratch_types=[
            pltpu.VMEM((g, k), jnp.int32),
            pltpu.VMEM((g, k), jnp.int32),
            pltpu.VMEM((k, C), jnp.float32),
            pltpu.VMEM((k, C), jnp.float32),
            pltpu.VMEM_SHARED((NPAD, C), jnp.float32),
            pltpu.SemaphoreType.DMA,
            pltpu.SemaphoreType.DMA,
            pltpu.SemaphoreType.DMA,
            pltpu.SemaphoreType.DMA,
        ],
        compiler_params=pltpu.CompilerParams(use_tc_tiling_on_sc=False),
    )(h, src2d, dst2d, zeros)


_BR = 2000
_GRID = N // _BR


def _phase_a_body(degp_ref, x_ref, w_ref, h_ref, dis_ref):
    deg = degp_ref[0, :, 0] + degp_ref[1, :, 0] + 1.0
    dis = lax.rsqrt(deg)[:, None]
    h_ref[...] = jnp.dot(x_ref[...] * dis, w_ref[...],
                         preferred_element_type=jnp.float32)
    dis_ref[...] = dis


def _phase_a(degp, x, w):
    return pl.pallas_call(
        _phase_a_body,
        grid=(_GRID,),
        in_specs=[
            pl.BlockSpec((NC, _BR, 16), lambda i: (0, i, 0)),
            pl.BlockSpec((_BR, F0), lambda i: (i, 0)),
            pl.BlockSpec((F0, F1), lambda i: (0, 0)),
        ],
        out_specs=[
            pl.BlockSpec((_BR, F1), lambda i: (i, 0)),
            pl.BlockSpec((_BR, 1), lambda i: (i, 0)),
        ],
        out_shape=[
            jax.ShapeDtypeStruct((NPAD, F1), jnp.float32),
            jax.ShapeDtypeStruct((N, 1), jnp.float32),
        ],
    )(degp, x, w)


def _phase_bc_body(acc_ref, dis_ref, b_ref, w_ref, hout_ref):
    dis = dis_ref[...]
    agg = acc_ref[0] + acc_ref[1]
    x = jnp.maximum(agg * dis + b_ref[...], 0.0)
    hout_ref[...] = jnp.dot(x * dis, w_ref[...],
                            preferred_element_type=jnp.float32)


def _phase_bc(acc, dis, b, w, Cin, Cout):
    return pl.pallas_call(
        _phase_bc_body,
        grid=(_GRID,),
        in_specs=[
            pl.BlockSpec((NC, _BR, Cin), lambda i: (0, i, 0)),
            pl.BlockSpec((_BR, 1), lambda i: (i, 0)),
            pl.BlockSpec((1, Cin), lambda i: (0, 0)),
            pl.BlockSpec((Cin, Cout), lambda i: (0, 0)),
        ],
        out_specs=pl.BlockSpec((_BR, Cout), lambda i: (i, 0)),
        out_shape=jax.ShapeDtypeStruct((NPAD, Cout), jnp.float32),
    )(acc, dis, b, w)


def _phase_d_body(acc_ref, dis_ref, b_ref, watt_ref, wfc_ref, bfc_ref,
                  wsc_ref, bsc_ref, score_ref, f_sc, colsum_sc, rep_sc):
    p = pl.program_id(0)
    i = pl.program_id(1)

    @pl.when(p == 0)
    def _():
        f = (acc_ref[0] + acc_ref[1]) * dis_ref[...] + b_ref[...]
        f_sc[pl.ds(i * _BR, _BR), :] = f

        @pl.when(i == 0)
        def _():
            colsum_sc[...] = jnp.zeros_like(colsum_sc)
            rep_sc[...] = jnp.zeros_like(rep_sc)

        colsum_sc[...] += jnp.sum(f, axis=0, keepdims=True)

    @pl.when(p == 1)
    def _():
        tg = jnp.tanh(jnp.dot(colsum_sc[...] * (1.0 / N), watt_ref[...],
                              preferred_element_type=jnp.float32))
        f = f_sc[pl.ds(i * _BR, _BR), :]
        sig = jax.nn.sigmoid(
            lax.dot_general(f, tg, (((1,), (1,)), ((), ())),
                            preferred_element_type=jnp.float32))
        rep_sc[...] += lax.dot_general(sig, f, (((0,), (0,)), ((), ())),
                                       preferred_element_type=jnp.float32)

        @pl.when(i == _GRID - 1)
        def _():
            scores = jnp.maximum(
                jnp.dot(rep_sc[...], wfc_ref[...],
                        preferred_element_type=jnp.float32) + bfc_ref[...],
                0.0)
            score_ref[...] = jax.nn.sigmoid(
                jnp.dot(scores, wsc_ref[...],
                        preferred_element_type=jnp.float32) + bsc_ref[...])


def _phase_d(acc, dis, b, watt, wfc, bfc, wsc, bsc):
    return pl.pallas_call(
        _phase_d_body,
        grid=(2, _GRID),
        in_specs=[
            pl.BlockSpec((NC, _BR, F3), lambda p, i: (0, i, 0)),
            pl.BlockSpec((_BR, 1), lambda p, i: (i, 0)),
            pl.BlockSpec((1, F3), lambda p, i: (0, 0)),
            pl.BlockSpec((F3, F3), lambda p, i: (0, 0)),
            pl.BlockSpec((F3, BN), lambda p, i: (0, 0)),
            pl.BlockSpec((1, BN), lambda p, i: (0, 0)),
            pl.BlockSpec((BN, 1), lambda p, i: (0, 0)),
            pl.BlockSpec((1, 1), lambda p, i: (0, 0)),
        ],
        out_specs=pl.BlockSpec((1, 1), lambda p, i: (0, 0)),
        out_shape=jax.ShapeDtypeStruct((1, 1), jnp.float32),
        scratch_shapes=[
            pltpu.VMEM((N, F3), jnp.float32),
            pltpu.VMEM((1, F3), jnp.float32),
            pltpu.VMEM((1, F3), jnp.float32),
        ],
    )(acc, dis, b, watt, wfc, bfc, wsc, bsc)


def kernel(features_1, edge_index_1, W1, b1, W2, b2, W3, b3, Watt, Wfc, bfc,
           Wsc, bsc):
    ones16 = jnp.ones((DEG_K, 16), jnp.float32)
    zeros16 = jnp.zeros((RPT, 16), jnp.float32)
    degp = _deg_pass(edge_index_1, ones16, zeros16)

    h1, dis = _phase_a(degp, features_1, W1)
    acc1 = _scatter_pass(h1, edge_index_1,
                         jnp.zeros((RPT, F1), jnp.float32), F1, 125, 16)
    h2 = _phase_bc(acc1, dis, b1.reshape(1, F1), W2, F1, F2)
    acc2 = _scatter_pass(h2, edge_index_1,
                         jnp.zeros((RPT, F2), jnp.float32), F2, 250, 8)
    h3 = _phase_bc(acc2, dis, b2.reshape(1, F2), W3, F2, F3)
    acc3 = _scatter_pass(h3, edge_index_1,
                         jnp.zeros((RPT, F3), jnp.float32), F3, 500, 4)
    score = _phase_d(acc3, dis, b3.reshape(1, F3), Watt, Wfc,
                     bfc.reshape(1, BN), Wsc, bsc.reshape(1, 1))
    return score


# speedup vs baseline: 1.1404x; 1.1404x over previous
import functools

import jax
import jax.numpy as jnp
from jax import lax
from jax.experimental import pallas as pl
from jax.experimental.pallas import tpu as pltpu
from jax.experimental.pallas import tpu_sc as plsc

N, E = 10000, 320000
NPAD = 10240
F0, F1, F2, F3, BN = 128, 128, 64, 32, 16

K = 125
CHUNKS = E // K
NC, NS = 2, 16
NW = NC * NS
CPW = CHUNKS // NW
RPT = NPAD // NS
G = 16

_MESH = plsc.VectorSubcoreMesh(core_axis_name="c", subcore_axis_name="s")


DEG_K = 500
DEG_CPW = (E // DEG_K) // NW


def _deg_body(dst_hbm, ones_hbm, zeros_hbm, out_hbm, idx_v, ones_v, acc_sh, sem):
    c = lax.axis_index("c")
    s = lax.axis_index("s")
    wid = c * NS + s
    pltpu.sync_copy(dst_hbm.at[pl.ds(wid * DEG_CPW, DEG_CPW)], idx_v)
    pltpu.sync_copy(ones_hbm, ones_v)
    pltpu.sync_copy(zeros_hbm, acc_sh.at[pl.ds(s * RPT, RPT)])
    plsc.subcore_barrier()

    def body(j, carry):
        pltpu.async_copy(ones_v, acc_sh.at[idx_v.at[j]], sem, add=True)
        return carry

    lax.fori_loop(0, DEG_CPW, body, 0)

    def drain(j, carry):
        pltpu.make_async_copy(ones_v, acc_sh.at[idx_v.at[0]], sem).wait()
        return carry

    lax.fori_loop(0, DEG_CPW, drain, 0)
    plsc.subcore_barrier()
    pltpu.sync_copy(acc_sh.at[pl.ds(s * RPT, RPT)],
                    out_hbm.at[c, pl.ds(s * RPT, RPT)])


def _deg_pass(edge_index, ones, zeros):
    dst2d = edge_index[1].reshape(E // DEG_K, DEG_K)
    return pl.kernel(
        _deg_body,
        out_type=jax.ShapeDtypeStruct((NC, NPAD, 16), jnp.float32),
        mesh=_MESH,
        scratch_types=[
            pltpu.VMEM((DEG_CPW, DEG_K), jnp.int32),
            pltpu.VMEM((DEG_K, 16), jnp.float32),
            pltpu.VMEM_SHARED((NPAD, 16), jnp.float32),
            pltpu.SemaphoreType.DMA,
        ],
        compiler_params=pltpu.CompilerParams(use_tc_tiling_on_sc=False),
    )(dst2d, ones, zeros)


def _scatter_body(k, g, cpw, h_hbm, src_hbm, dst_hbm, zeros_hbm, out_hbm,
                  srcv, dstv, b0, b1, acc_sh, g0, g1):
    c = lax.axis_index("c")
    s = lax.axis_index("s")
    wid = c * NS + s

    @pl.when(c == 0)
    def _():
        pltpu.sync_copy(h_hbm.at[pl.ds(s * RPT, RPT)],
                        acc_sh.at[pl.ds(s * RPT, RPT)])

    @pl.when(c == 1)
    def _():
        pltpu.sync_copy(zeros_hbm, acc_sh.at[pl.ds(s * RPT, RPT)])

    plsc.subcore_barrier()

    def group(gi, carry):
        base = wid * cpw + gi * g
        pltpu.sync_copy(src_hbm.at[pl.ds(base, g)], srcv)
        pltpu.sync_copy(dst_hbm.at[pl.ds(base, g)], dstv)
        pltpu.async_copy(h_hbm.at[srcv.at[0]], b0, g0)
        pltpu.async_copy(h_hbm.at[srcv.at[1]], b1, g1)

        def step(u, carry2):
            k = 2 * u
            pltpu.make_async_copy(h_hbm.at[srcv.at[k]], b0, g0).wait()
            pltpu.sync_copy(b0, acc_sh.at[dstv.at[k]], add=True)

            @pl.when(k + 2 < g)
            def _():
                pltpu.async_copy(h_hbm.at[srcv.at[k + 2]], b0, g0)

            pltpu.make_async_copy(h_hbm.at[srcv.at[k + 1]], b1, g1).wait()
            pltpu.sync_copy(b1, acc_sh.at[dstv.at[k + 1]], add=True)

            @pl.when(k + 3 < g)
            def _():
                pltpu.async_copy(h_hbm.at[srcv.at[k + 3]], b1, g1)

            return carry2

        lax.fori_loop(0, g // 2, step, 0)
        return carry

    lax.fori_loop(0, cpw // g, group, 0)
    plsc.subcore_barrier()
    pltpu.sync_copy(acc_sh.at[pl.ds(s * RPT, RPT)],
                    out_hbm.at[c, pl.ds(s * RPT, RPT)])


def _scatter_pass(h, edge_index, zeros, C, k, g):
    chunks = E // k
    cpw = chunks // NW
    src2d = edge_index[0].reshape(chunks, k)
    dst2d = edge_index[1].reshape(chunks, k)
    return pl.kernel(
        functools.partial(_scatter_body, k, g, cpw),
        out_type=jax.ShapeDtypeStruct((NC, NPAD, C), jnp.float32),
        mesh=_MESH,
        scratch_types=[
            pltpu.VMEM((g, k), jnp.int32),
            pltpu.VMEM((g, k), jnp.int32),
            pltpu.VMEM((k, C), jnp.float32),
            pltpu.VMEM((k, C), jnp.float32),
            pltpu.VMEM_SHARED((NPAD, C), jnp.float32),
            pltpu.SemaphoreType.DMA,
            pltpu.SemaphoreType.DMA,
        ],
        compiler_params=pltpu.CompilerParams(use_tc_tiling_on_sc=False),
    )(h, src2d, dst2d, zeros)


_BR = 2000
_GRID = N // _BR


def _phase_a_body(degp_ref, x_ref, w_ref, h_ref, dis_ref):
    deg = degp_ref[0, :, 0] + degp_ref[1, :, 0] + 1.0
    dis = lax.rsqrt(deg)[:, None]
    h_ref[...] = jnp.dot(x_ref[...] * dis, w_ref[...],
                         preferred_element_type=jnp.float32)
    dis_ref[...] = dis


def _phase_a(degp, x, w):
    return pl.pallas_call(
        _phase_a_body,
        grid=(_GRID,),
        in_specs=[
            pl.BlockSpec((NC, _BR, 16), lambda i: (0, i, 0)),
            pl.BlockSpec((_BR, F0), lambda i: (i, 0)),
            pl.BlockSpec((F0, F1), lambda i: (0, 0)),
        ],
        out_specs=[
            pl.BlockSpec((_BR, F1), lambda i: (i, 0)),
            pl.BlockSpec((_BR, 1), lambda i: (i, 0)),
        ],
        out_shape=[
            jax.ShapeDtypeStruct((NPAD, F1), jnp.float32),
            jax.ShapeDtypeStruct((N, 1), jnp.float32),
        ],
    )(degp, x, w)


def _phase_bc_body(acc_ref, dis_ref, b_ref, w_ref, hout_ref):
    dis = dis_ref[...]
    agg = acc_ref[0] + acc_ref[1]
    x = jnp.maximum(agg * dis + b_ref[...], 0.0)
    hout_ref[...] = jnp.dot(x * dis, w_ref[...],
                            preferred_element_type=jnp.float32)


def _phase_bc(acc, dis, b, w, Cin, Cout):
    return pl.pallas_call(
        _phase_bc_body,
        grid=(_GRID,),
        in_specs=[
            pl.BlockSpec((NC, _BR, Cin), lambda i: (0, i, 0)),
            pl.BlockSpec((_BR, 1), lambda i: (i, 0)),
            pl.BlockSpec((1, Cin), lambda i: (0, 0)),
            pl.BlockSpec((Cin, Cout), lambda i: (0, 0)),
        ],
        out_specs=pl.BlockSpec((_BR, Cout), lambda i: (i, 0)),
        out_shape=jax.ShapeDtypeStruct((NPAD, Cout), jnp.float32),
    )(acc, dis, b, w)


def _phase_d_body(acc_ref, dis_ref, b_ref, watt_ref, wfc_ref, bfc_ref,
                  wsc_ref, bsc_ref, score_ref, f_sc, colsum_sc, rep_sc):
    p = pl.program_id(0)
    i = pl.program_id(1)

    @pl.when(p == 0)
    def _():
        f = (acc_ref[0] + acc_ref[1]) * dis_ref[...] + b_ref[...]
        f_sc[pl.ds(i * _BR, _BR), :] = f

        @pl.when(i == 0)
        def _():
            colsum_sc[...] = jnp.zeros_like(colsum_sc)
            rep_sc[...] = jnp.zeros_like(rep_sc)

        colsum_sc[...] += jnp.sum(f, axis=0, keepdims=True)

    @pl.when(p == 1)
    def _():
        tg = jnp.tanh(jnp.dot(colsum_sc[...] * (1.0 / N), watt_ref[...],
                              preferred_element_type=jnp.float32))
        f = f_sc[pl.ds(i * _BR, _BR), :]
        sig = jax.nn.sigmoid(
            lax.dot_general(f, tg, (((1,), (1,)), ((), ())),
                            preferred_element_type=jnp.float32))
        rep_sc[...] += lax.dot_general(sig, f, (((0,), (0,)), ((), ())),
                                       preferred_element_type=jnp.float32)

        @pl.when(i == _GRID - 1)
        def _():
            scores = jnp.maximum(
                jnp.dot(rep_sc[...], wfc_ref[...],
                        preferred_element_type=jnp.float32) + bfc_ref[...],
                0.0)
            score_ref[...] = jax.nn.sigmoid(
                jnp.dot(scores, wsc_ref[...],
                        preferred_element_type=jnp.float32) + bsc_ref[...])


def _phase_d(acc, dis, b, watt, wfc, bfc, wsc, bsc):
    return pl.pallas_call(
        _phase_d_body,
        grid=(2, _GRID),
        in_specs=[
            pl.BlockSpec((NC, _BR, F3), lambda p, i: (0, i, 0)),
            pl.BlockSpec((_BR, 1), lambda p, i: (i, 0)),
            pl.BlockSpec((1, F3), lambda p, i: (0, 0)),
            pl.BlockSpec((F3, F3), lambda p, i: (0, 0)),
            pl.BlockSpec((F3, BN), lambda p, i: (0, 0)),
            pl.BlockSpec((1, BN), lambda p, i: (0, 0)),
            pl.BlockSpec((BN, 1), lambda p, i: (0, 0)),
            pl.BlockSpec((1, 1), lambda p, i: (0, 0)),
        ],
        out_specs=pl.BlockSpec((1, 1), lambda p, i: (0, 0)),
        out_shape=jax.ShapeDtypeStruct((1, 1), jnp.float32),
        scratch_shapes=[
            pltpu.VMEM((N, F3), jnp.float32),
            pltpu.VMEM((1, F3), jnp.float32),
            pltpu.VMEM((1, F3), jnp.float32),
        ],
    )(acc, dis, b, watt, wfc, bfc, wsc, bsc)


def kernel(features_1, edge_index_1, W1, b1, W2, b2, W3, b3, Watt, Wfc, bfc,
           Wsc, bsc):
    ones16 = jnp.ones((DEG_K, 16), jnp.float32)
    zeros16 = jnp.zeros((RPT, 16), jnp.float32)
    degp = _deg_pass(edge_index_1, ones16, zeros16)

    h1, dis = _phase_a(degp, features_1, W1)
    acc1 = _scatter_pass(h1, edge_index_1,
                         jnp.zeros((RPT, F1), jnp.float32), F1, 125, 16)
    h2 = _phase_bc(acc1, dis, b1.reshape(1, F1), W2, F1, F2)
    acc2 = _scatter_pass(h2, edge_index_1,
                         jnp.zeros((RPT, F2), jnp.float32), F2, 250, 8)
    h3 = _phase_bc(acc2, dis, b2.reshape(1, F2), W3, F2, F3)
    acc3 = _scatter_pass(h3, edge_index_1,
                         jnp.zeros((RPT, F3), jnp.float32), F3, 500, 4)
    score = _phase_d(acc3, dis, b3.reshape(1, F3), Watt, Wfc,
                     bfc.reshape(1, BN), Wsc, bsc.reshape(1, 1))
    return score


# speedup vs baseline: 1.2270x; 1.0759x over previous
import functools

import jax
import jax.numpy as jnp
from jax import lax
from jax.experimental import pallas as pl
from jax.experimental.pallas import tpu as pltpu
from jax.experimental.pallas import tpu_sc as plsc

N, E = 10000, 320000
NPAD = 10240
F0, F1, F2, F3, BN = 128, 128, 64, 32, 16

K = 125
CHUNKS = E // K
NC, NS = 2, 16
NW = NC * NS
CPW = CHUNKS // NW
RPT = NPAD // NS
G = 16

_MESH = plsc.VectorSubcoreMesh(core_axis_name="c", subcore_axis_name="s")


DEG_K = 500
DEG_CPW = (E // DEG_K) // NW


def _deg_body(dst_hbm, ones_hbm, zeros_hbm, out_hbm, idx_v, ones_v, acc_sh, sem):
    c = lax.axis_index("c")
    s = lax.axis_index("s")
    wid = c * NS + s
    pltpu.sync_copy(dst_hbm.at[pl.ds(wid * DEG_CPW, DEG_CPW)], idx_v)
    pltpu.sync_copy(ones_hbm, ones_v)
    pltpu.sync_copy(zeros_hbm, acc_sh.at[pl.ds(s * RPT, RPT)])
    plsc.subcore_barrier()

    def body(j, carry):
        pltpu.async_copy(ones_v, acc_sh.at[idx_v.at[j]], sem, add=True)
        return carry

    lax.fori_loop(0, DEG_CPW, body, 0)

    def drain(j, carry):
        pltpu.make_async_copy(ones_v, acc_sh.at[idx_v.at[0]], sem).wait()
        return carry

    lax.fori_loop(0, DEG_CPW, drain, 0)
    plsc.subcore_barrier()
    pltpu.sync_copy(acc_sh.at[pl.ds(s * RPT, RPT)],
                    out_hbm.at[c, pl.ds(s * RPT, RPT)])


def _deg_pass(edge_index, ones, zeros):
    dst2d = edge_index[1].reshape(E // DEG_K, DEG_K)
    return pl.kernel(
        _deg_body,
        out_type=jax.ShapeDtypeStruct((NC, NPAD, 16), jnp.float32),
        mesh=_MESH,
        scratch_types=[
            pltpu.VMEM((DEG_CPW, DEG_K), jnp.int32),
            pltpu.VMEM((DEG_K, 16), jnp.float32),
            pltpu.VMEM_SHARED((NPAD, 16), jnp.float32),
            pltpu.SemaphoreType.DMA,
        ],
        compiler_params=pltpu.CompilerParams(use_tc_tiling_on_sc=False),
    )(dst2d, ones, zeros)


def _scatter_body(k, g, cpw, h_hbm, src_hbm, dst_hbm, zeros_hbm, out_hbm,
                  srcv, dstv, b0, b1, acc_sh, g0, g1):
    c = lax.axis_index("c")
    s = lax.axis_index("s")
    wid = c * NS + s

    @pl.when(c == 0)
    def _():
        pltpu.sync_copy(h_hbm.at[pl.ds(s * RPT, RPT)],
                        acc_sh.at[pl.ds(s * RPT, RPT)])

    @pl.when(c == 1)
    def _():
        pltpu.sync_copy(zeros_hbm, acc_sh.at[pl.ds(s * RPT, RPT)])

    plsc.subcore_barrier()

    def group(gi, carry):
        base = wid * cpw + gi * g
        pltpu.sync_copy(src_hbm.at[pl.ds(base, g)], srcv)
        pltpu.sync_copy(dst_hbm.at[pl.ds(base, g)], dstv)
        pltpu.async_copy(h_hbm.at[srcv.at[0]], b0, g0)
        pltpu.async_copy(h_hbm.at[srcv.at[1]], b1, g1)

        def step(u, carry2):
            k = 2 * u
            pltpu.make_async_copy(h_hbm.at[srcv.at[k]], b0, g0).wait()
            pltpu.sync_copy(b0, acc_sh.at[dstv.at[k]], add=True)

            @pl.when(k + 2 < g)
            def _():
                pltpu.async_copy(h_hbm.at[srcv.at[k + 2]], b0, g0)

            pltpu.make_async_copy(h_hbm.at[srcv.at[k + 1]], b1, g1).wait()
            pltpu.sync_copy(b1, acc_sh.at[dstv.at[k + 1]], add=True)

            @pl.when(k + 3 < g)
            def _():
                pltpu.async_copy(h_hbm.at[srcv.at[k + 3]], b1, g1)

            return carry2

        lax.fori_loop(0, g // 2, step, 0)
        return carry

    lax.fori_loop(0, cpw // g, group, 0)
    plsc.subcore_barrier()
    pltpu.sync_copy(acc_sh.at[pl.ds(s * RPT, RPT)],
                    out_hbm.at[c, pl.ds(s * RPT, RPT)])


def _scatter_body4(k, cpw, h_hbm, src_hbm, dst_hbm, zeros_hbm, out_hbm,
                   srcv, dstv, b0, b1, b2, b3, acc_sh, g0, g1, g2, g3):
    c = lax.axis_index("c")
    s = lax.axis_index("s")
    wid = c * NS + s
    pltpu.sync_copy(src_hbm.at[pl.ds(wid * cpw, cpw)], srcv)
    pltpu.sync_copy(dst_hbm.at[pl.ds(wid * cpw, cpw)], dstv)

    @pl.when(c == 0)
    def _():
        pltpu.sync_copy(h_hbm.at[pl.ds(s * RPT, RPT)],
                        acc_sh.at[pl.ds(s * RPT, RPT)])

    @pl.when(c == 1)
    def _():
        pltpu.sync_copy(zeros_hbm, acc_sh.at[pl.ds(s * RPT, RPT)])

    plsc.subcore_barrier()

    pltpu.async_copy(h_hbm.at[srcv.at[0]], b0, g0)
    pltpu.async_copy(h_hbm.at[srcv.at[1]], b1, g1)
    pltpu.async_copy(h_hbm.at[srcv.at[2]], b2, g2)
    pltpu.async_copy(h_hbm.at[srcv.at[3]], b3, g3)

    def _slot(j, buf, gsem):
        pltpu.make_async_copy(h_hbm.at[srcv.at[j]], buf, gsem).wait()
        pltpu.sync_copy(buf, acc_sh.at[dstv.at[j]], add=True)

        @pl.when(j + 4 < cpw)
        def _():
            pltpu.async_copy(h_hbm.at[srcv.at[j + 4]], buf, gsem)

    def step(u, carry):
        j = 4 * u
        _slot(j, b0, g0)
        _slot(j + 1, b1, g1)
        _slot(j + 2, b2, g2)
        _slot(j + 3, b3, g3)
        return carry

    lax.fori_loop(0, cpw // 4, step, 0)
    plsc.subcore_barrier()
    pltpu.sync_copy(acc_sh.at[pl.ds(s * RPT, RPT)],
                    out_hbm.at[c, pl.ds(s * RPT, RPT)])


def _scatter_pass4(h, edge_index, zeros, C, k):
    chunks = E // k
    cpw = chunks // NW
    src2d = edge_index[0].reshape(chunks, k)
    dst2d = edge_index[1].reshape(chunks, k)
    return pl.kernel(
        functools.partial(_scatter_body4, k, cpw),
        out_type=jax.ShapeDtypeStruct((NC, NPAD, C), jnp.float32),
        mesh=_MESH,
        scratch_types=[
            pltpu.VMEM((cpw, k), jnp.int32),
            pltpu.VMEM((cpw, k), jnp.int32),
            pltpu.VMEM((k, C), jnp.float32),
            pltpu.VMEM((k, C), jnp.float32),
            pltpu.VMEM((k, C), jnp.float32),
            pltpu.VMEM((k, C), jnp.float32),
            pltpu.VMEM_SHARED((NPAD, C), jnp.float32),
            pltpu.SemaphoreType.DMA,
            pltpu.SemaphoreType.DMA,
            pltpu.SemaphoreType.DMA,
            pltpu.SemaphoreType.DMA,
        ],
        compiler_params=pltpu.CompilerParams(use_tc_tiling_on_sc=False),
    )(h, src2d, dst2d, zeros)


def _scatter_pass(h, edge_index, zeros, C, k, g):
    chunks = E // k
    cpw = chunks // NW
    src2d = edge_index[0].reshape(chunks, k)
    dst2d = edge_index[1].reshape(chunks, k)
    return pl.kernel(
        functools.partial(_scatter_body, k, g, cpw),
        out_type=jax.ShapeDtypeStruct((NC, NPAD, C), jnp.float32),
        mesh=_MESH,
        scratch_types=[
            pltpu.VMEM((g, k), jnp.int32),
            pltpu.VMEM((g, k), jnp.int32),
            pltpu.VMEM((k, C), jnp.float32),
            pltpu.VMEM((k, C), jnp.float32),
            pltpu.VMEM_SHARED((NPAD, C), jnp.float32),
            pltpu.SemaphoreType.DMA,
            pltpu.SemaphoreType.DMA,
        ],
        compiler_params=pltpu.CompilerParams(use_tc_tiling_on_sc=False),
    )(h, src2d, dst2d, zeros)


_BR = 2000
_GRID = N // _BR


def _phase_a_body(degp_ref, x_ref, w_ref, h_ref, dis_ref):
    deg = degp_ref[0, :, 0] + degp_ref[1, :, 0] + 1.0
    dis = lax.rsqrt(deg)[:, None]
    h_ref[...] = jnp.dot(x_ref[...] * dis, w_ref[...],
                         preferred_element_type=jnp.float32)
    dis_ref[...] = dis


def _phase_a(degp, x, w):
    return pl.pallas_call(
        _phase_a_body,
        grid=(_GRID,),
        in_specs=[
            pl.BlockSpec((NC, _BR, 16), lambda i: (0, i, 0)),
            pl.BlockSpec((_BR, F0), lambda i: (i, 0)),
            pl.BlockSpec((F0, F1), lambda i: (0, 0)),
        ],
        out_specs=[
            pl.BlockSpec((_BR, F1), lambda i: (i, 0)),
            pl.BlockSpec((_BR, 1), lambda i: (i, 0)),
        ],
        out_shape=[
            jax.ShapeDtypeStruct((NPAD, F1), jnp.float32),
            jax.ShapeDtypeStruct((N, 1), jnp.float32),
        ],
    )(degp, x, w)


def _phase_bc_body(acc_ref, dis_ref, b_ref, w_ref, hout_ref):
    dis = dis_ref[...]
    agg = acc_ref[0] + acc_ref[1]
    x = jnp.maximum(agg * dis + b_ref[...], 0.0)
    hout_ref[...] = jnp.dot(x * dis, w_ref[...],
                            preferred_element_type=jnp.float32)


def _phase_bc(acc, dis, b, w, Cin, Cout):
    return pl.pallas_call(
        _phase_bc_body,
        grid=(_GRID,),
        in_specs=[
            pl.BlockSpec((NC, _BR, Cin), lambda i: (0, i, 0)),
            pl.BlockSpec((_BR, 1), lambda i: (i, 0)),
            pl.BlockSpec((1, Cin), lambda i: (0, 0)),
            pl.BlockSpec((Cin, Cout), lambda i: (0, 0)),
        ],
        out_specs=pl.BlockSpec((_BR, Cout), lambda i: (i, 0)),
        out_shape=jax.ShapeDtypeStruct((NPAD, Cout), jnp.float32),
    )(acc, dis, b, w)


def _phase_d_body(acc_ref, dis_ref, b_ref, watt_ref, wfc_ref, bfc_ref,
                  wsc_ref, bsc_ref, score_ref, f_sc, colsum_sc, rep_sc):
    p = pl.program_id(0)
    i = pl.program_id(1)

    @pl.when(p == 0)
    def _():
        f = (acc_ref[0] + acc_ref[1]) * dis_ref[...] + b_ref[...]
        f_sc[pl.ds(i * _BR, _BR), :] = f

        @pl.when(i == 0)
        def _():
            colsum_sc[...] = jnp.zeros_like(colsum_sc)
            rep_sc[...] = jnp.zeros_like(rep_sc)

        colsum_sc[...] += jnp.sum(f, axis=0, keepdims=True)

    @pl.when(p == 1)
    def _():
        tg = jnp.tanh(jnp.dot(colsum_sc[...] * (1.0 / N), watt_ref[...],
                              preferred_element_type=jnp.float32))
        f = f_sc[pl.ds(i * _BR, _BR), :]
        sig = jax.nn.sigmoid(
            lax.dot_general(f, tg, (((1,), (1,)), ((), ())),
                            preferred_element_type=jnp.float32))
        rep_sc[...] += lax.dot_general(sig, f, (((0,), (0,)), ((), ())),
                                       preferred_element_type=jnp.float32)

        @pl.when(i == _GRID - 1)
        def _():
            scores = jnp.maximum(
                jnp.dot(rep_sc[...], wfc_ref[...],
                        preferred_element_type=jnp.float32) + bfc_ref[...],
                0.0)
            score_ref[...] = jax.nn.sigmoid(
                jnp.dot(scores, wsc_ref[...],
                        preferred_element_type=jnp.float32) + bsc_ref[...])


def _phase_d(acc, dis, b, watt, wfc, bfc, wsc, bsc):
    return pl.pallas_call(
        _phase_d_body,
        grid=(2, _GRID),
        in_specs=[
            pl.BlockSpec((NC, _BR, F3), lambda p, i: (0, i, 0)),
            pl.BlockSpec((_BR, 1), lambda p, i: (i, 0)),
            pl.BlockSpec((1, F3), lambda p, i: (0, 0)),
            pl.BlockSpec((F3, F3), lambda p, i: (0, 0)),
            pl.BlockSpec((F3, BN), lambda p, i: (0, 0)),
            pl.BlockSpec((1, BN), lambda p, i: (0, 0)),
            pl.BlockSpec((BN, 1), lambda p, i: (0, 0)),
            pl.BlockSpec((1, 1), lambda p, i: (0, 0)),
        ],
        out_specs=pl.BlockSpec((1, 1), lambda p, i: (0, 0)),
        out_shape=jax.ShapeDtypeStruct((1, 1), jnp.float32),
        scratch_shapes=[
            pltpu.VMEM((N, F3), jnp.float32),
            pltpu.VMEM((1, F3), jnp.float32),
            pltpu.VMEM((1, F3), jnp.float32),
        ],
    )(acc, dis, b, watt, wfc, bfc, wsc, bsc)


def kernel(features_1, edge_index_1, W1, b1, W2, b2, W3, b3, Watt, Wfc, bfc,
           Wsc, bsc):
    ones16 = jnp.ones((DEG_K, 16), jnp.float32)
    zeros16 = jnp.zeros((RPT, 16), jnp.float32)
    degp = _deg_pass(edge_index_1, ones16, zeros16)

    h1, dis = _phase_a(degp, features_1, W1)
    acc1 = _scatter_pass(h1, edge_index_1,
                         jnp.zeros((RPT, F1), jnp.float32), F1, 125, 16)
    h2 = _phase_bc(acc1, dis, b1.reshape(1, F1), W2, F1, F2)
    acc2 = _scatter_pass4(h2, edge_index_1,
                          jnp.zeros((RPT, F2), jnp.float32), F2, 250)
    h3 = _phase_bc(acc2, dis, b2.reshape(1, F2), W3, F2, F3)
    acc3 = _scatter_pass4(h3, edge_index_1,
                          jnp.zeros((RPT, F3), jnp.float32), F3, 500)
    score = _phase_d(acc3, dis, b3.reshape(1, F3), Watt, Wfc,
                     bfc.reshape(1, BN), Wsc, bsc.reshape(1, 1))
    return score
